# Initial kernel scaffold; baseline (speedup 1.0000x reference)
import jax
import jax.numpy as jnp
from jax.experimental import pallas as pl


def kernel(pc, normal, m1a_w1, m1a_w2, m1a_xr, m1a_cr, m1b_w1, m1b_w2, m1b_xr, m1b_cr, m2a_w1, m2a_w2, m2a_cr, m2b_w1, m2b_w2, m2b_cr, m3a_w1, m3a_w2, m3a_cr, m3b_w1, m3b_w2, m3b_cr, m4_w, fc1, fc2, fc3):
    raise NotImplementedError("write your pallas kernel here")


# speedup vs baseline: 4.0164x; 4.0164x over previous
import functools

import jax
import jax.numpy as jnp
from jax.experimental import pallas as pl


def _fps_kernel(xx_ref, xy_ref, xz_ref, ox_ref, oy_ref, oz_ref, *, npoint):
    xx = xx_ref[...]
    xy = xy_ref[...]
    xz = xz_ref[...]
    B, N = xx.shape
    lane = jax.lax.broadcasted_iota(jnp.int32, (B, N), 1)
    olane = jax.lax.broadcasted_iota(jnp.int32, (B, npoint), 1)

    def body(i, carry):
        dists, idx, ax, ay, az = carry
        oh = lane == idx
        cx = jnp.sum(jnp.where(oh, xx, 0.0), axis=1, keepdims=True)
        cy = jnp.sum(jnp.where(oh, xy, 0.0), axis=1, keepdims=True)
        cz = jnp.sum(jnp.where(oh, xz, 0.0), axis=1, keepdims=True)
        slot = olane == i
        ax = jnp.where(slot, cx, ax)
        ay = jnp.where(slot, cy, ay)
        az = jnp.where(slot, cz, az)
        d = (xx - cx) ** 2 + (xy - cy) ** 2 + (xz - cz) ** 2
        dists = jnp.minimum(dists, d)
        idx = jnp.argmax(dists, axis=1).astype(jnp.int32)[:, None]
        return dists, idx, ax, ay, az

    dists0 = jnp.full((B, N), 1e10, jnp.float32)
    idx0 = jnp.zeros((B, 1), jnp.int32)
    z = jnp.zeros((B, npoint), jnp.float32)
    _, _, ax, ay, az = jax.lax.fori_loop(0, npoint, body, (dists0, idx0, z, z, z))
    ox_ref[...] = ax
    oy_ref[...] = ay
    oz_ref[...] = az


def _fps(xx, xy, xz, npoint):
    B, _ = xx.shape
    out = jax.ShapeDtypeStruct((B, npoint), jnp.float32)
    return pl.pallas_call(
        functools.partial(_fps_kernel, npoint=npoint),
        out_shape=[out, out, out],
    )(xx, xy, xz)


def _cumsum_lanes(x):
    n = x.shape[-1]
    s = 1
    while s < n:
        pad = jnp.zeros_like(x[:, :s])
        x = x + jnp.concatenate([pad, x[:, :-s]], axis=1)
        s *= 2
    return x


def _rsconv_kernel(nxx_ref, nxy_ref, nxz_ref, xx_ref, xy_ref, xz_ref,
                   f_ref, w1_ref, w2_ref, wcr_ref, o_ref,
                   *, r2, ns, nc, xr_mode):
    cx = nxx_ref[0, 0, 0][:, None]
    cy = nxy_ref[0, 0, 0][:, None]
    cz = nxz_ref[0, 0, 0][:, None]
    xx = xx_ref[0, 0][None, :]
    xy = xy_ref[0, 0][None, :]
    xz = xz_ref[0, 0][None, :]
    Pb = cx.shape[0]
    N = xx.shape[1]

    d2 = (cx - xx) ** 2 + (cy - xy) ** 2 + (cz - xz) ** 2
    mask = d2 <= r2
    rank = _cumsum_lanes(mask.astype(jnp.int32))
    selm = jnp.where(mask & (rank <= ns), 1.0, 0.0)
    dist = jnp.sqrt(d2 + 1e-12)

    w1 = w1_ref[...]
    wd = w1[0:1, :]
    wc = w1[1:4, :] - w1[7:10, :]
    wp = w1[4:7, :] + w1[7:10, :]

    cmat = jnp.concatenate([cx, cy, cz], axis=1)
    xmat = jnp.concatenate([xx_ref[0, 0][:, None], xy_ref[0, 0][:, None],
                            xz_ref[0, 0][:, None]], axis=1)
    A = jnp.dot(cmat, wc, preferred_element_type=jnp.float32)
    Bn = jnp.dot(xmat, wp, preferred_element_type=jnp.float32)

    if xr_mode:
        F = jnp.maximum(jnp.dot(xmat, f_ref[...],
                                preferred_element_type=jnp.float32), 0.0)
    else:
        F = f_ref[0]
    C = F.shape[1]

    w2 = w2_ref[...]
    m = jnp.full((Pb, C), 0.0, jnp.float32)
    for j in range(N // nc):
        sl = slice(j * nc, (j + 1) * nc)
        zj = jnp.maximum(dist[:, sl][:, :, None] * wd[None, :, :]
                         + A[:, None, :] + Bn[sl][None, :, :], 0.0)
        wj = jnp.dot(zj.reshape(Pb * nc, w2.shape[0]), w2,
                     preferred_element_type=jnp.float32).reshape(Pb, nc, C)
        xj = jnp.maximum(wj * F[sl][None, :, :], 0.0)
        xj = xj * selm[:, sl][:, :, None]
        m = jnp.maximum(m, jnp.max(xj, axis=1))

    o_ref[0] = jnp.maximum(jnp.dot(m, wcr_ref[...],
                                   preferred_element_type=jnp.float32), 0.0)


def _rsconv(nx, x, feats, w1, w2, wcr, wxr, radius, ns, pb, nc):
    (nxx, nxy, nxz) = nx
    (xx, xy, xz) = x
    B, P = nxx.shape
    N = xx.shape[1]
    cout = wcr.shape[1]
    xr_mode = feats is None
    if xr_mode:
        f_arg = wxr
        f_spec = pl.BlockSpec(wxr.shape, lambda b, p: (0, 0))
    else:
        f_arg = feats
        f_spec = pl.BlockSpec((1, N, feats.shape[2]), lambda b, p: (b, 0, 0))
    grid = (B, P // pb)
    nxx = nxx.reshape(B, P // pb, 1, pb)
    nxy = nxy.reshape(B, P // pb, 1, pb)
    nxz = nxz.reshape(B, P // pb, 1, pb)
    xx = xx.reshape(B, 1, N)
    xy = xy.reshape(B, 1, N)
    xz = xz.reshape(B, 1, N)
    plane_nx = pl.BlockSpec((1, 1, 1, pb), lambda b, p: (b, p, 0, 0))
    plane_x = pl.BlockSpec((1, 1, N), lambda b, p: (b, 0, 0))
    wspec = lambda w: pl.BlockSpec(w.shape, lambda b, p: (0, 0))
    return pl.pallas_call(
        functools.partial(_rsconv_kernel, r2=radius * radius, ns=ns, nc=nc,
                          xr_mode=xr_mode),
        grid=grid,
        in_specs=[plane_nx, plane_nx, plane_nx, plane_x, plane_x, plane_x,
                  f_spec, wspec(w1), wspec(w2), wspec(wcr)],
        out_specs=pl.BlockSpec((1, pb, cout), lambda b, p: (b, p, 0)),
        out_shape=jax.ShapeDtypeStruct((B, P, cout), jnp.float32),
    )(nxx, nxy, nxz, xx, xy, xz, f_arg, w1, w2, wcr)


def _head_kernel(f_ref, m4_ref, fc1_ref, fc2_ref, fc3_ref, o_ref):
    x = f_ref[0]
    y = jnp.maximum(jnp.dot(x, m4_ref[...],
                            preferred_element_type=jnp.float32), 0.0)
    m = jnp.mean(y, axis=0, keepdims=True)
    h1 = jnp.maximum(jnp.dot(m, fc1_ref[...],
                             preferred_element_type=jnp.float32), 0.0)
    h2 = jnp.maximum(jnp.dot(h1, fc2_ref[...],
                             preferred_element_type=jnp.float32), 0.0)
    o_ref[0] = jnp.dot(h2, fc3_ref[...],
                       preferred_element_type=jnp.float32)


def _head(feats, m4_w, fc1, fc2, fc3):
    B, P, C = feats.shape
    wspec = lambda w: pl.BlockSpec(w.shape, lambda b: (0, 0))
    out = pl.pallas_call(
        _head_kernel,
        grid=(B,),
        in_specs=[pl.BlockSpec((1, P, C), lambda b: (b, 0, 0)),
                  wspec(m4_w), wspec(fc1), wspec(fc2), wspec(fc3)],
        out_specs=pl.BlockSpec((1, 1, fc3.shape[1]), lambda b: (b, 0, 0)),
        out_shape=jax.ShapeDtypeStruct((B, 1, fc3.shape[1]), jnp.float32),
    )(feats, m4_w, fc1, fc2, fc3)
    return out[:, 0, :]


def kernel(pc, normal, m1a_w1, m1a_w2, m1a_xr, m1a_cr, m1b_w1, m1b_w2, m1b_xr,
           m1b_cr, m2a_w1, m2a_w2, m2a_cr, m2b_w1, m2b_w2, m2b_cr, m3a_w1,
           m3a_w2, m3a_cr, m3b_w1, m3b_w2, m3b_cr, m4_w, fc1, fc2, fc3):
    del normal
    x0 = (pc[..., 0], pc[..., 1], pc[..., 2])

    nx1 = _fps(*x0, 512)
    f1a = _rsconv(nx1, x0, None, m1a_w1, m1a_w2, m1a_cr, m1a_xr,
                  0.15, 24, pb=16, nc=512)
    f1b = _rsconv(nx1, x0, None, m1b_w1, m1b_w2, m1b_cr, m1b_xr,
                  0.23, 48, pb=16, nc=512)
    feats1 = jnp.concatenate([f1a, f1b], axis=-1)

    nx2 = _fps(*nx1, 256)
    f2a = _rsconv(nx2, nx1, feats1, m2a_w1, m2a_w2, m2a_cr, None,
                  0.2, 32, pb=8, nc=256)
    f2b = _rsconv(nx2, nx1, feats1, m2b_w1, m2b_w2, m2b_cr, None,
                  0.32, 64, pb=8, nc=256)
    feats2 = jnp.concatenate([f2a, f2b], axis=-1)

    nx3 = _fps(*nx2, 128)
    f3a = _rsconv(nx3, nx2, feats2, m3a_w1, m3a_w2, m3a_cr, None,
                  0.2, 32, pb=8, nc=256)
    f3b = _rsconv(nx3, nx2, feats2, m3b_w1, m3b_w2, m3b_cr, None,
                  0.32, 64, pb=8, nc=256)
    feats3 = jnp.concatenate([f3a, f3b], axis=-1)

    return _head(feats3, m4_w, fc1, fc2, fc3)


# speedup vs baseline: 5.3552x; 1.3333x over previous
import functools

import jax
import jax.numpy as jnp
from jax.experimental import pallas as pl


def _fps_kernel(xx_ref, xy_ref, xz_ref, ox_ref, oy_ref, oz_ref, *, npoint):
    xx = xx_ref[...]
    xy = xy_ref[...]
    xz = xz_ref[...]
    B, N = xx.shape
    lane = jax.lax.broadcasted_iota(jnp.int32, (B, N), 1)
    olane = jax.lax.broadcasted_iota(jnp.int32, (B, npoint), 1)

    def body(i, carry):
        dists, idx, ax, ay, az = carry
        oh = lane == idx
        cx = jnp.sum(jnp.where(oh, xx, 0.0), axis=1, keepdims=True)
        cy = jnp.sum(jnp.where(oh, xy, 0.0), axis=1, keepdims=True)
        cz = jnp.sum(jnp.where(oh, xz, 0.0), axis=1, keepdims=True)
        slot = olane == i
        ax = jnp.where(slot, cx, ax)
        ay = jnp.where(slot, cy, ay)
        az = jnp.where(slot, cz, az)
        d = (xx - cx) ** 2 + (xy - cy) ** 2 + (xz - cz) ** 2
        dists = jnp.minimum(dists, d)
        idx = jnp.argmax(dists, axis=1).astype(jnp.int32)[:, None]
        return dists, idx, ax, ay, az

    dists0 = jnp.full((B, N), 1e10, jnp.float32)
    idx0 = jnp.zeros((B, 1), jnp.int32)
    z = jnp.zeros((B, npoint), jnp.float32)
    _, _, ax, ay, az = jax.lax.fori_loop(0, npoint, body, (dists0, idx0, z, z, z))
    ox_ref[...] = ax
    oy_ref[...] = ay
    oz_ref[...] = az


def _fps(xx, xy, xz, npoint):
    B, _ = xx.shape
    out = jax.ShapeDtypeStruct((B, npoint), jnp.float32)
    return pl.pallas_call(
        functools.partial(_fps_kernel, npoint=npoint),
        out_shape=[out, out, out],
    )(xx, xy, xz)


def _cumsum_lanes(x):
    n = x.shape[-1]
    s = 1
    while s < n:
        pad = jnp.zeros_like(x[:, :s])
        x = x + jnp.concatenate([pad, x[:, :-s]], axis=1)
        s *= 2
    return x


def _rsconv_kernel(nxx_ref, nxy_ref, nxz_ref, xx_ref, xy_ref, xz_ref,
                   f_ref, w1a_ref, w1b_ref, w2_ref, wcr_ref, o_ref,
                   *, r2a, r2b, nsa, nsb, nc, xr_mode):
    cx = nxx_ref[0, 0, 0][:, None]
    cy = nxy_ref[0, 0, 0][:, None]
    cz = nxz_ref[0, 0, 0][:, None]
    xx = xx_ref[0, 0][None, :]
    xy = xy_ref[0, 0][None, :]
    xz = xz_ref[0, 0][None, :]
    Pb = cx.shape[0]
    N = xx.shape[1]

    d2 = (cx - xx) ** 2 + (cy - xy) ** 2 + (cz - xz) ** 2
    mask_a = d2 <= r2a
    mask_b = d2 <= r2b
    rank_a = _cumsum_lanes(mask_a.astype(jnp.int32))
    rank_b = _cumsum_lanes(mask_b.astype(jnp.int32))
    selm_a = jnp.where(mask_a & (rank_a <= nsa), 1.0, 0.0)
    selm_b = jnp.where(mask_b & (rank_b <= nsb), 1.0, 0.0)
    dist = jnp.sqrt(d2 + 1e-12)

    w1 = jnp.concatenate([w1a_ref[...], w1b_ref[...]], axis=1)
    M2 = w1.shape[1]
    wd = w1[0:1, :]
    wc = w1[1:4, :] - w1[7:10, :]
    wp = w1[4:7, :] + w1[7:10, :]

    cmat = jnp.concatenate([cx, cy, cz], axis=1)
    xmat = jnp.concatenate([xx_ref[0, 0][:, None], xy_ref[0, 0][:, None],
                            xz_ref[0, 0][:, None]], axis=1)
    A = jnp.dot(cmat, wc, preferred_element_type=jnp.float32)
    Bn = jnp.dot(xmat, wp, preferred_element_type=jnp.float32)

    if xr_mode:
        F = jnp.maximum(jnp.dot(xmat, f_ref[...],
                                preferred_element_type=jnp.float32), 0.0)
    else:
        F0 = f_ref[0]
        F = jnp.concatenate([F0, F0], axis=1)
    C = F.shape[1]

    mid_iota = jax.lax.broadcasted_iota(jnp.int32, (1, 1, M2), 2)
    cha = jnp.where(mid_iota < M2 // 2, 1.0, 0.0)
    chb = 1.0 - cha

    w2 = w2_ref[...]
    m = jnp.full((Pb, C), 0.0, jnp.float32)
    for j in range(N // nc):
        sl = slice(j * nc, (j + 1) * nc)
        zm = selm_a[:, sl][:, :, None] * cha + selm_b[:, sl][:, :, None] * chb
        zj = jnp.maximum(dist[:, sl][:, :, None] * wd[None, :, :]
                         + A[:, None, :] + Bn[sl][None, :, :], 0.0) * zm
        wj = jnp.dot(zj.reshape(Pb * nc, M2), w2,
                     preferred_element_type=jnp.float32).reshape(Pb, nc, C)
        xj = jnp.maximum(wj * F[sl][None, :, :], 0.0)
        m = jnp.maximum(m, jnp.max(xj, axis=1))

    o_ref[0] = jnp.maximum(jnp.dot(m, wcr_ref[...],
                                   preferred_element_type=jnp.float32), 0.0)


def _blockdiag(a, b):
    z = jnp.zeros((a.shape[0] + b.shape[0], a.shape[1] + b.shape[1]),
                  jnp.float32)
    z = z.at[:a.shape[0], :a.shape[1]].set(a)
    return z.at[a.shape[0]:, a.shape[1]:].set(b)


def _rsconv(nx, x, feats, wa, wb, radii, nss, pb, nc):
    (nxx, nxy, nxz) = nx
    (xx, xy, xz) = x
    B, P = nxx.shape
    N = xx.shape[1]
    w1a, w2a, wcra, xra = wa
    w1b, w2b, wcrb, xrb = wb
    w2 = _blockdiag(w2a, w2b)
    wcr = _blockdiag(wcra, wcrb)
    cout = wcr.shape[1]
    xr_mode = feats is None
    if xr_mode:
        f_arg = jnp.concatenate([xra, xrb], axis=1)
        f_spec = pl.BlockSpec(f_arg.shape, lambda b, p: (0, 0))
    else:
        f_arg = feats
        f_spec = pl.BlockSpec((1, N, feats.shape[2]), lambda b, p: (b, 0, 0))
    grid = (B, P // pb)
    nxx = nxx.reshape(B, P // pb, 1, pb)
    nxy = nxy.reshape(B, P // pb, 1, pb)
    nxz = nxz.reshape(B, P // pb, 1, pb)
    xx = xx.reshape(B, 1, N)
    xy = xy.reshape(B, 1, N)
    xz = xz.reshape(B, 1, N)
    plane_nx = pl.BlockSpec((1, 1, 1, pb), lambda b, p: (b, p, 0, 0))
    plane_x = pl.BlockSpec((1, 1, N), lambda b, p: (b, 0, 0))
    wspec = lambda w: pl.BlockSpec(w.shape, lambda b, p: (0, 0))
    return pl.pallas_call(
        functools.partial(_rsconv_kernel, r2a=radii[0] * radii[0],
                          r2b=radii[1] * radii[1], nsa=nss[0], nsb=nss[1],
                          nc=nc, xr_mode=xr_mode),
        grid=grid,
        in_specs=[plane_nx, plane_nx, plane_nx, plane_x, plane_x, plane_x,
                  f_spec, wspec(w1a), wspec(w1b), wspec(w2), wspec(wcr)],
        out_specs=pl.BlockSpec((1, pb, cout), lambda b, p: (b, p, 0)),
        out_shape=jax.ShapeDtypeStruct((B, P, cout), jnp.float32),
    )(nxx, nxy, nxz, xx, xy, xz, f_arg, w1a, w1b, w2, wcr)


def _head_kernel(f_ref, m4_ref, fc1_ref, fc2_ref, fc3_ref, o_ref):
    x = f_ref[0]
    y = jnp.maximum(jnp.dot(x, m4_ref[...],
                            preferred_element_type=jnp.float32), 0.0)
    m = jnp.mean(y, axis=0, keepdims=True)
    h1 = jnp.maximum(jnp.dot(m, fc1_ref[...],
                             preferred_element_type=jnp.float32), 0.0)
    h2 = jnp.maximum(jnp.dot(h1, fc2_ref[...],
                             preferred_element_type=jnp.float32), 0.0)
    o_ref[0] = jnp.dot(h2, fc3_ref[...],
                       preferred_element_type=jnp.float32)


def _head(feats, m4_w, fc1, fc2, fc3):
    B, P, C = feats.shape
    wspec = lambda w: pl.BlockSpec(w.shape, lambda b: (0, 0))
    out = pl.pallas_call(
        _head_kernel,
        grid=(B,),
        in_specs=[pl.BlockSpec((1, P, C), lambda b: (b, 0, 0)),
                  wspec(m4_w), wspec(fc1), wspec(fc2), wspec(fc3)],
        out_specs=pl.BlockSpec((1, 1, fc3.shape[1]), lambda b: (b, 0, 0)),
        out_shape=jax.ShapeDtypeStruct((B, 1, fc3.shape[1]), jnp.float32),
    )(feats, m4_w, fc1, fc2, fc3)
    return out[:, 0, :]


def kernel(pc, normal, m1a_w1, m1a_w2, m1a_xr, m1a_cr, m1b_w1, m1b_w2, m1b_xr,
           m1b_cr, m2a_w1, m2a_w2, m2a_cr, m2b_w1, m2b_w2, m2b_cr, m3a_w1,
           m3a_w2, m3a_cr, m3b_w1, m3b_w2, m3b_cr, m4_w, fc1, fc2, fc3):
    del normal
    x0 = (pc[..., 0], pc[..., 1], pc[..., 2])

    nx1 = _fps(*x0, 512)
    feats1 = _rsconv(nx1, x0, None,
                     (m1a_w1, m1a_w2, m1a_cr, m1a_xr),
                     (m1b_w1, m1b_w2, m1b_cr, m1b_xr),
                     (0.15, 0.23), (24, 48), pb=32, nc=256)

    nx2 = _fps(*nx1, 256)
    feats2 = _rsconv(nx2, nx1, feats1,
                     (m2a_w1, m2a_w2, m2a_cr, None),
                     (m2b_w1, m2b_w2, m2b_cr, None),
                     (0.2, 0.32), (32, 64), pb=8, nc=256)

    nx3 = _fps(*nx2, 128)
    feats3 = _rsconv(nx3, nx2, feats2,
                     (m3a_w1, m3a_w2, m3a_cr, None),
                     (m3b_w1, m3b_w2, m3b_cr, None),
                     (0.2, 0.32), (32, 64), pb=8, nc=128)

    return _head(feats3, m4_w, fc1, fc2, fc3)
